# baseline (speedup 1.0000x reference)
import jax
import jax.numpy as jnp
from jax.experimental import pallas as pl

H = 128
GROUPS = 16
PER_GROUP = 8
SLOTS = GROUPS * PER_GROUP
B = 4096
ROW = SLOTS * H
OUT = GROUPS * H

BR = 64
NBLK = B // BR


def _tc_body(x_ref, acc_ref, o_ref):
    i = pl.program_id(0)
    blk = jnp.max(x_ref[...].reshape(BR // 8, 8, ROW), axis=0)

    @pl.when(i == 0)
    def _init():
        acc_ref[...] = blk

    @pl.when(i > 0)
    def _accum():
        acc_ref[...] = jnp.maximum(acc_ref[...], blk)

    @pl.when(i == NBLK - 1)
    def _final():
        a = jnp.max(acc_ref[...], axis=0)
        a = jnp.max(a.reshape(PER_GROUP, GROUPS, H), axis=0)
        o_ref[...] = a.reshape(1, OUT)


def kernel(x):
    _, out = pl.pallas_call(
        _tc_body,
        grid=(NBLK,),
        in_specs=[pl.BlockSpec((BR, ROW), lambda i: (i, 0))],
        out_specs=[
            pl.BlockSpec((8, ROW), lambda i: (0, 0)),
            pl.BlockSpec((1, OUT), lambda i: (0, 0)),
        ],
        out_shape=[
            jax.ShapeDtypeStruct((8, ROW), jnp.float32),
            jax.ShapeDtypeStruct((1, OUT), jnp.float32),
        ],
    )(x)
    return out


# speedup vs baseline: 1.0636x; 1.0636x over previous
import jax
import jax.numpy as jnp
from jax.experimental import pallas as pl

H = 128
GROUPS = 16
PER_GROUP = 8
SLOTS = GROUPS * PER_GROUP
B = 4096
ROW = SLOTS * H
OUT = GROUPS * H

BR = 128
NBLK = B // BR


def _tc_body(x_ref, acc_ref, o_ref):
    i = pl.program_id(0)
    blk = jnp.max(x_ref[...].reshape(BR // 8, 8, ROW), axis=0)

    @pl.when(i == 0)
    def _init():
        acc_ref[...] = blk

    @pl.when(i > 0)
    def _accum():
        acc_ref[...] = jnp.maximum(acc_ref[...], blk)

    @pl.when(i == NBLK - 1)
    def _final():
        a = jnp.max(acc_ref[...], axis=0)
        a = jnp.max(a.reshape(PER_GROUP, GROUPS, H), axis=0)
        o_ref[...] = a.reshape(1, OUT)


def kernel(x):
    _, out = pl.pallas_call(
        _tc_body,
        grid=(NBLK,),
        in_specs=[pl.BlockSpec((BR, ROW), lambda i: (i, 0))],
        out_specs=[
            pl.BlockSpec((8, ROW), lambda i: (0, 0)),
            pl.BlockSpec((1, OUT), lambda i: (0, 0)),
        ],
        out_shape=[
            jax.ShapeDtypeStruct((8, ROW), jnp.float32),
            jax.ShapeDtypeStruct((1, OUT), jnp.float32),
        ],
    )(x)
    return out


# speedup vs baseline: 1.0739x; 1.0097x over previous
import jax
import jax.numpy as jnp
from jax import lax
from jax.experimental import pallas as pl
from jax.experimental.pallas import tpu as pltpu

H = 128
GROUPS = 16
PER_GROUP = 8
SLOTS = GROUPS * PER_GROUP
B = 4096
ROW = SLOTS * H
OUT = GROUPS * H

BR = 64
NBLK = B // BR
NB = 4


def _body(x_hbm, o_ref, bufs, acc, sems):
    def _start(t, b):
        pltpu.make_async_copy(
            x_hbm.at[pl.ds(t * BR, BR), :], bufs.at[b], sems[b]
        ).start()

    def _wait(b):
        pltpu.make_async_copy(
            x_hbm.at[pl.ds(0, BR), :], bufs.at[b], sems[b]
        ).wait()

    def _fold(b):
        return jnp.max(bufs[b].reshape(BR // 8, 8, ROW), axis=0)

    for b in range(NB):
        _start(b, b)

    _wait(0)
    acc[...] = _fold(0)
    _start(NB, 0)
    for b in range(1, NB):
        _wait(b)
        acc[...] = jnp.maximum(acc[...], _fold(b))
        _start(NB + b, b)

    def _loop(k, _):
        t = NB + k * NB
        for b in range(NB):
            _wait(b)
            acc[...] = jnp.maximum(acc[...], _fold(b))

            @pl.when(t + NB + b < NBLK)
            def _next():
                _start(t + NB + b, b)

        return 0

    lax.fori_loop(0, NBLK // NB - 1, _loop, 0)

    a = jnp.max(acc[...], axis=0)
    a = jnp.max(a.reshape(PER_GROUP, GROUPS, H), axis=0)
    o_ref[...] = a.reshape(1, OUT)


def kernel(x):
    return pl.pallas_call(
        _body,
        in_specs=[pl.BlockSpec(memory_space=pl.ANY)],
        out_specs=pl.BlockSpec(memory_space=pltpu.MemorySpace.VMEM),
        out_shape=jax.ShapeDtypeStruct((1, OUT), jnp.float32),
        scratch_shapes=[
            pltpu.VMEM((NB, BR, ROW), jnp.float32),
            pltpu.VMEM((8, ROW), jnp.float32),
            [pltpu.SemaphoreType.DMA] * NB,
        ],
    )(x)
